# baseline (speedup 1.0000x reference)
import jax
import jax.numpy as jnp
from jax import lax
from jax.experimental import pallas as pl
from jax.experimental.pallas import tpu as pltpu
from jax.experimental.pallas import tpu_sc as plsc

NC, NS, LN = 2, 16, 16


def _make_edge_pass(cfg, with_hist):
    n = cfg["n"]
    CH = cfg["ch"]
    SUB = cfg["sub"]
    NSUB = CH // SUB
    T_EDGE = cfg["t_edge"]
    T_HIST = T_EDGE * NC
    NCH_L = T_EDGE // CH
    NCH_H = T_HIST // CH
    HBINS = cfg["hbins"]
    HROWS = cfg["hrows"]
    HB_T = HBINS // NS
    HR_T = HROWS // NS
    ZF = cfg["zf"]
    ZR = cfg["zr"]
    emb = cfg["emb"]

    mesh = plsc.VectorSubcoreMesh(
        core_axis_name="c", subcore_axis_name="s", num_cores=NC,
        num_subcores=NS)

    e_tot = NC * NS * T_EDGE

    if with_hist:
        out_type = [
            jax.ShapeDtypeStruct((NC, HROWS, emb), jnp.float32),
            jax.ShapeDtypeStruct((e_tot,), jnp.float32),
        ]
    else:
        out_type = [jax.ShapeDtypeStruct((NC, HROWS, emb), jnp.float32)]

    scratch_types = [
        pltpu.VMEM_SHARED((HROWS, emb), jnp.float32),
        pltpu.VMEM((ZR, emb), jnp.float32),
        pltpu.VMEM((2, CH, 3), jnp.int32),
        pltpu.VMEM((2, CH), jnp.float32),
        pltpu.VMEM((2, NSUB, SUB), jnp.int32),
        pltpu.VMEM((2, NSUB, SUB), jnp.int32),
        pltpu.VMEM((2, CH, emb), jnp.float32),
        pltpu.SemaphoreType.DMA,
        pltpu.SemaphoreType.DMA,
        pltpu.SemaphoreType.DMA,
    ]
    if with_hist:
        scratch_types += [
            pltpu.VMEM_SHARED((HBINS,), jnp.float32),
            pltpu.VMEM((ZF,), jnp.float32),
            pltpu.VMEM((128,), jnp.float32),
            pltpu.VMEM((2, NSUB, SUB), jnp.int32),
            pltpu.SemaphoreType.DMA,
            pltpu.SemaphoreType.DMA,
        ]

    def body(*refs):
        if with_hist:
            (tri_h, tab_h, part_h, vals_h,
             acc, zrows, tri2, vals2, idxw2, idxf2, rows2,
             sem_l, sem_g, sem_s,
             hist, zflat, ones, idxv2, sem_h, sem_v) = refs
        else:
            (tri_h, vals_in_h, tab_h, part_h,
             acc, zrows, tri2, vals2, idxw2, idxf2, rows2,
             sem_l, sem_g, sem_s) = refs

        cid = lax.axis_index("c")
        sid = lax.axis_index("s")
        wid = cid * NS + sid
        lanes = lax.iota(jnp.int32, LN)

        def zr_body(i, _):
            zrows[i] = jnp.zeros((LN,), jnp.float32)
            return 0
        lax.fori_loop(0, ZR, zr_body, 0)
        for k in range(HR_T // ZR):
            pltpu.async_copy(zrows, acc.at[pl.ds(sid * HR_T + k * ZR, ZR)],
                             sem_s)

        if with_hist:
            def zf_body(i, _):
                zflat[pl.ds(i * LN, LN)] = jnp.zeros((LN,), jnp.float32)
                return 0
            lax.fori_loop(0, ZF // LN, zf_body, 0)
            for i in range(128 // LN):
                ones[pl.ds(i * LN, LN)] = jnp.ones((LN,), jnp.float32)
            for k in range(HB_T // ZF):
                pltpu.async_copy(
                    zflat, hist.at[pl.ds(sid * HB_T + k * ZF, ZF)], sem_g)
            for k in range(HB_T // ZF):
                pltpu.make_async_copy(
                    zflat, hist.at[pl.ds(sid * HB_T + k * ZF, ZF)],
                    sem_g).wait()
        for k in range(HR_T // ZR):
            pltpu.make_async_copy(
                zrows, acc.at[pl.ds(sid * HR_T + k * ZR, ZR)], sem_s).wait()

        plsc.subcore_barrier()

        def load_tri(k, b, base):
            pltpu.async_copy(tri_h.at[pl.ds(base + k * CH, CH)],
                             tri2.at[b], sem_l)

        def wait_tri(b):
            pltpu.make_async_copy(tri_h.at[pl.ds(0, CH)],
                                  tri2.at[b], sem_l).wait()

        def get_col(b, j, col):
            bs = jnp.full((LN,), b, jnp.int32)
            cs = jnp.full((LN,), col, jnp.int32)
            return plsc.load_gather(tri2, [bs, j * LN + lanes, cs])

        if with_hist:
            hbase = sid * T_HIST
            load_tri(0, 0, hbase)

            def hist_chunk(k, _):
                b = lax.rem(k, 2)
                @pl.when(k >= 2)
                def _():
                    for r in range(NSUB):
                        pltpu.make_async_copy(
                            ones.at[pl.ds(0, SUB)],
                            hist.at[idxv2.at[b, r]], sem_s).wait()
                wait_tri(b)

                @pl.when(k + 1 < NCH_H)
                def _():
                    load_tri(k + 1, 1 - b, hbase)

                def vec_body(j, _):
                    r, col = j // (SUB // LN), (j % (SUB // LN)) * LN
                    fj = get_col(b, j, 0)
                    pj = get_col(b, j, 1)
                    idxv2[b, r, pl.ds(col, LN)] = pj * n + fj
                    return 0
                lax.fori_loop(0, CH // LN, vec_body, 0)
                for r in range(NSUB):
                    pltpu.async_copy(ones.at[pl.ds(0, SUB)],
                                     hist.at[idxv2.at[b, r]], sem_s, add=True)
                return 0
            lax.fori_loop(0, NCH_H, hist_chunk, 0)
            for b in range(2):
                for r in range(NSUB):
                    pltpu.make_async_copy(
                        ones.at[pl.ds(0, SUB)],
                        hist.at[idxv2.at[b, r]], sem_s).wait()
            plsc.subcore_barrier()

        ebase = wid * T_EDGE
        load_tri(0, 0, ebase)
        if not with_hist:
            pltpu.async_copy(vals_in_h.at[pl.ds(ebase, CH)],
                             vals2.at[0], sem_l)

        def edge_chunk(k, _):
            b = lax.rem(k, 2)
            @pl.when(k >= 2)
            def _():
                for r in range(NSUB):
                    pltpu.make_async_copy(
                        rows2.at[b, pl.ds(r * SUB, SUB)],
                        acc.at[idxf2.at[b, r]], sem_s).wait()
                if with_hist:
                    pltpu.make_async_copy(
                        vals2.at[b], vals_h.at[pl.ds(0, CH)], sem_v).wait()
            wait_tri(b)
            if not with_hist:
                pltpu.make_async_copy(vals_in_h.at[pl.ds(0, CH)],
                                      vals2.at[b], sem_l).wait()

            @pl.when(k + 1 < NCH_L)
            def _():
                load_tri(k + 1, 1 - b, ebase)
                if not with_hist:
                    pltpu.async_copy(
                        vals_in_h.at[pl.ds(ebase + (k + 1) * CH, CH)],
                        vals2.at[1 - b], sem_l)

            bs = jnp.full((LN,), b, jnp.int32)

            def vec_body(j, _):
                r = j // (SUB // LN)
                col = (j % (SUB // LN)) * LN
                fj = get_col(b, j, 0)
                pj = get_col(b, j, 1)
                tj = get_col(b, j, 2)
                idxw2[b, r, pl.ds(col, LN)] = pj * n + tj
                idxf2[b, r, pl.ds(col, LN)] = fj
                if with_hist:
                    idxv2[b, r, pl.ds(col, LN)] = pj * n + fj
                return 0
            lax.fori_loop(0, CH // LN, vec_body, 0)

            for r in range(NSUB):
                pltpu.async_copy(tab_h.at[idxw2.at[b, r]],
                                 rows2.at[b, pl.ds(r * SUB, SUB)], sem_g)

            if with_hist:
                for r in range(NSUB):
                    pltpu.async_copy(hist.at[idxv2.at[b, r]],
                                     vals2.at[b, pl.ds(r * SUB, SUB)], sem_h)
                for r in range(NSUB):
                    pltpu.make_async_copy(
                        hist.at[idxv2.at[b, r]],
                        vals2.at[b, pl.ds(r * SUB, SUB)], sem_h).wait()

                def inv_body(j, _):
                    v = vals2[b, pl.ds(j * LN, LN)]
                    vals2[b, pl.ds(j * LN, LN)] = 1.0 / v
                    return 0
                lax.fori_loop(0, CH // LN, inv_body, 0)
                pltpu.async_copy(vals2.at[b],
                                 vals_h.at[pl.ds(ebase + k * CH, CH)], sem_v)

            for r in range(NSUB):
                pltpu.make_async_copy(
                    tab_h.at[idxw2.at[b, r]],
                    rows2.at[b, pl.ds(r * SUB, SUB)], sem_g).wait()

            def scale_body(j, _):
                sp = plsc.load_gather(vals2, [bs, jnp.full((LN,), j,
                                                           jnp.int32)])
                rows2[b, j] = rows2[b, j] * sp
                return 0
            lax.fori_loop(0, CH, scale_body, 0)

            for r in range(NSUB):
                pltpu.async_copy(rows2.at[b, pl.ds(r * SUB, SUB)],
                                 acc.at[idxf2.at[b, r]], sem_s, add=True)
            return 0
        lax.fori_loop(0, NCH_L, edge_chunk, 0)
        for b in range(2):
            for r in range(NSUB):
                pltpu.make_async_copy(
                    rows2.at[b, pl.ds(r * SUB, SUB)],
                    acc.at[idxf2.at[b, r]], sem_s).wait()
            if with_hist:
                pltpu.make_async_copy(
                    vals2.at[b], vals_h.at[pl.ds(0, CH)], sem_v).wait()

        plsc.subcore_barrier()

        pltpu.sync_copy(acc.at[pl.ds(sid * HR_T, HR_T)],
                        part_h.at[cid].at[pl.ds(sid * HR_T, HR_T)])

    kern = pl.kernel(body, out_type=out_type, mesh=mesh,
                     scratch_types=scratch_types,
                     compiler_params=pltpu.CompilerParams(
                         use_tc_tiling_on_sc=False,
                         needs_layout_passes=False))
    return kern


def _tc_hw2(n_real, r, emb, ncls, blk):
    grid = n_real // blk

    def body(hp_ref, b1_ref, w2_ref, out_ref):
        h = jax.nn.relu(hp_ref[0] + hp_ref[1] + b1_ref[0][None, :])
        for p in range(r):
            out_ref[p] = jnp.dot(h, w2_ref[p],
                                 preferred_element_type=jnp.float32)

    return pl.pallas_call(
        body,
        grid=(grid,),
        in_specs=[
            pl.BlockSpec((NC, blk, emb), lambda i: (0, i, 0)),
            pl.BlockSpec((1, emb), lambda i: (0, 0)),
            pl.BlockSpec((r, emb, ncls), lambda i: (0, 0, 0)),
        ],
        out_specs=pl.BlockSpec((r, blk, ncls), lambda i: (0, i, 0)),
        out_shape=jax.ShapeDtypeStruct((r, n_real, ncls), jnp.float32),
    )


def _tc_combine(n_real, ncls, blk):
    grid = n_real // blk

    def body(op_ref, b2_ref, out_ref):
        out_ref[...] = op_ref[0] + op_ref[1] + b2_ref[0][None, :]

    return pl.pallas_call(
        body,
        grid=(grid,),
        in_specs=[
            pl.BlockSpec((NC, blk, ncls), lambda i: (0, i, 0)),
            pl.BlockSpec((1, ncls), lambda i: (0, 0)),
        ],
        out_specs=pl.BlockSpec((blk, ncls), lambda i: (i, 0)),
        out_shape=jax.ShapeDtypeStruct((n_real, ncls), jnp.float32),
    )


def _rgcn(triples, weights1, weights2, bias1, bias2, cfg):
    n, r = cfg["n"], cfg["r"]
    emb, ncls = cfg["emb"], cfg["ncls"]

    w1_flat = weights1.reshape(r * n, emb)

    hpart, vals = _make_edge_pass(cfg, with_hist=True)(triples, w1_flat)

    hw2 = _tc_hw2(n, r, emb, ncls, cfg["tc_blk"])(
        hpart, bias1.reshape(1, emb), weights2)
    hw2_flat = hw2.reshape(r * n, ncls)

    (opart,) = _make_edge_pass(cfg, with_hist=False)(triples, vals, hw2_flat)

    out = _tc_combine(n, ncls, cfg["tc_blk"])(opart, bias2.reshape(1, ncls))
    return out


_CFG_FULL = dict(
    n=50000, r=8, emb=16, ncls=16,
    ch=400, sub=80, t_edge=50000,
    hbins=409600, hrows=50000,
    zf=3200, zr=125,
    tc_blk=2000,
)


def kernel(triples, weights1, weights2, bias1, bias2):
    return _rgcn(triples, weights1, weights2, bias1, bias2, _CFG_FULL)


# speedup vs baseline: 4.6968x; 4.6968x over previous
import jax
import jax.numpy as jnp
from jax import lax
from jax.experimental import pallas as pl
from jax.experimental.pallas import tpu as pltpu
from jax.experimental.pallas import tpu_sc as plsc

NC, NS, LN = 2, 16, 16


def _make_edge_pass(cfg, with_hist):
    n = cfg["n"]
    CH = cfg["ch"]
    SUB = cfg["sub"]
    NSUB = CH // SUB
    T_EDGE = cfg["t_edge"]
    T_HIST = T_EDGE * NC
    NCH_L = T_EDGE // CH
    NCH_H = T_HIST // CH
    HBINS = cfg["hbins"]
    HROWS = cfg["hrows"]
    HB_T = HBINS // NS
    HR_T = HROWS // NS
    ZF = cfg["zf"]
    ZR = cfg["zr"]
    emb = cfg["emb"]

    mesh = plsc.VectorSubcoreMesh(
        core_axis_name="c", subcore_axis_name="s", num_cores=NC,
        num_subcores=NS)

    e_tot = NC * NS * T_EDGE

    if with_hist:
        out_type = [
            jax.ShapeDtypeStruct((NC, HROWS, emb), jnp.float32),
            jax.ShapeDtypeStruct((e_tot,), jnp.float32),
        ]
    else:
        out_type = [jax.ShapeDtypeStruct((NC, HROWS, emb), jnp.float32)]

    scratch_types = [
        pltpu.VMEM_SHARED((HROWS, emb), jnp.float32),
        pltpu.VMEM((ZR, emb), jnp.float32),
        pltpu.VMEM((2, CH), jnp.int32),
        pltpu.VMEM((2, CH), jnp.int32),
        pltpu.VMEM((2, CH), jnp.int32),
        pltpu.VMEM((2, CH), jnp.float32),
        pltpu.VMEM((2, NSUB, SUB), jnp.int32),
        pltpu.VMEM((2, NSUB, SUB), jnp.int32),
        pltpu.VMEM((2, CH, emb), jnp.float32),
        pltpu.SemaphoreType.DMA,
        pltpu.SemaphoreType.DMA,
        pltpu.SemaphoreType.DMA,
    ]
    if with_hist:
        scratch_types += [
            pltpu.VMEM_SHARED((HBINS,), jnp.float32),
            pltpu.VMEM((ZF,), jnp.float32),
            pltpu.VMEM((128,), jnp.float32),
            pltpu.VMEM((2, NSUB, SUB), jnp.int32),
            pltpu.SemaphoreType.DMA,
            pltpu.SemaphoreType.DMA,
        ]

    def body(*refs):
        if with_hist:
            (fr_h, p_h, to_h, tab_h, part_h, vals_h,
             acc, zrows, fr2, p2, to2, vals2, idxw2, idxf2, rows2,
             sem_l, sem_g, sem_s,
             hist, zflat, ones, idxv2, sem_h, sem_v) = refs
        else:
            (fr_h, p_h, to_h, vals_in_h, tab_h, part_h,
             acc, zrows, fr2, p2, to2, vals2, idxw2, idxf2, rows2,
             sem_l, sem_g, sem_s) = refs

        cid = lax.axis_index("c")
        sid = lax.axis_index("s")
        wid = cid * NS + sid
        lanes = lax.iota(jnp.int32, LN)

        def zr_body(i, _):
            zrows[i] = jnp.zeros((LN,), jnp.float32)
            return 0
        lax.fori_loop(0, ZR, zr_body, 0)
        for k in range(HR_T // ZR):
            pltpu.async_copy(zrows, acc.at[pl.ds(sid * HR_T + k * ZR, ZR)],
                             sem_s)

        if with_hist:
            def zf_body(i, _):
                zflat[pl.ds(i * LN, LN)] = jnp.zeros((LN,), jnp.float32)
                return 0
            lax.fori_loop(0, ZF // LN, zf_body, 0)
            for i in range(128 // LN):
                ones[pl.ds(i * LN, LN)] = jnp.ones((LN,), jnp.float32)
            for k in range(HB_T // ZF):
                pltpu.async_copy(
                    zflat, hist.at[pl.ds(sid * HB_T + k * ZF, ZF)], sem_g)
            for k in range(HB_T // ZF):
                pltpu.make_async_copy(
                    zflat, hist.at[pl.ds(sid * HB_T + k * ZF, ZF)],
                    sem_g).wait()
        for k in range(HR_T // ZR):
            pltpu.make_async_copy(
                zrows, acc.at[pl.ds(sid * HR_T + k * ZR, ZR)], sem_s).wait()

        plsc.subcore_barrier()

        def load_tri(k, b, base, need_to):
            srcs = [fr_h, p_h] + ([to_h] if need_to else [])
            dsts = [fr2, p2] + ([to2] if need_to else [])
            for s, d in zip(srcs, dsts):
                pltpu.async_copy(s.at[pl.ds(base + k * CH, CH)],
                                 d.at[b], sem_l)

        def wait_tri(b, need_to):
            srcs = [fr_h, p_h] + ([to_h] if need_to else [])
            dsts = [fr2, p2] + ([to2] if need_to else [])
            for s, d in zip(srcs, dsts):
                pltpu.make_async_copy(s.at[pl.ds(0, CH)],
                                      d.at[b], sem_l).wait()

        if with_hist:
            hbase = sid * T_HIST
            load_tri(0, 0, hbase, False)

            def hist_chunk(k, _):
                b = lax.rem(k, 2)
                @pl.when(k >= 2)
                def _():
                    for r in range(NSUB):
                        pltpu.make_async_copy(
                            ones.at[pl.ds(0, SUB)],
                            hist.at[idxv2.at[b, r]], sem_s).wait()
                wait_tri(b, False)

                @pl.when(k + 1 < NCH_H)
                def _():
                    load_tri(k + 1, 1 - b, hbase, False)

                def vec_body(j, _):
                    r, col = j // (SUB // LN), (j % (SUB // LN)) * LN
                    fj = fr2[b, pl.ds(j * LN, LN)]
                    pj = p2[b, pl.ds(j * LN, LN)]
                    idxv2[b, r, pl.ds(col, LN)] = pj * n + fj
                    return 0
                lax.fori_loop(0, CH // LN, vec_body, 0)
                for r in range(NSUB):
                    pltpu.async_copy(ones.at[pl.ds(0, SUB)],
                                     hist.at[idxv2.at[b, r]], sem_s, add=True)
                return 0
            lax.fori_loop(0, NCH_H, hist_chunk, 0)
            for b in range(2):
                for r in range(NSUB):
                    pltpu.make_async_copy(
                        ones.at[pl.ds(0, SUB)],
                        hist.at[idxv2.at[b, r]], sem_s).wait()
            plsc.subcore_barrier()

        ebase = wid * T_EDGE
        load_tri(0, 0, ebase, True)
        if not with_hist:
            pltpu.async_copy(vals_in_h.at[pl.ds(ebase, CH)],
                             vals2.at[0], sem_l)

        def edge_chunk(k, _):
            b = lax.rem(k, 2)
            @pl.when(k >= 2)
            def _():
                for r in range(NSUB):
                    pltpu.make_async_copy(
                        rows2.at[b, pl.ds(r * SUB, SUB)],
                        acc.at[idxf2.at[b, r]], sem_s).wait()
                if with_hist:
                    pltpu.make_async_copy(
                        vals2.at[b], vals_h.at[pl.ds(0, CH)], sem_v).wait()
            wait_tri(b, True)
            if not with_hist:
                pltpu.make_async_copy(vals_in_h.at[pl.ds(0, CH)],
                                      vals2.at[b], sem_l).wait()

            @pl.when(k + 1 < NCH_L)
            def _():
                load_tri(k + 1, 1 - b, ebase, True)
                if not with_hist:
                    pltpu.async_copy(
                        vals_in_h.at[pl.ds(ebase + (k + 1) * CH, CH)],
                        vals2.at[1 - b], sem_l)

            bs = jnp.full((LN,), b, jnp.int32)

            def vec_body(j, _):
                r = j // (SUB // LN)
                col = (j % (SUB // LN)) * LN
                fj = fr2[b, pl.ds(j * LN, LN)]
                pj = p2[b, pl.ds(j * LN, LN)]
                tj = to2[b, pl.ds(j * LN, LN)]
                idxw2[b, r, pl.ds(col, LN)] = pj * n + tj
                idxf2[b, r, pl.ds(col, LN)] = fj
                if with_hist:
                    idxv2[b, r, pl.ds(col, LN)] = pj * n + fj
                return 0
            lax.fori_loop(0, CH // LN, vec_body, 0)

            for r in range(NSUB):
                pltpu.async_copy(tab_h.at[idxw2.at[b, r]],
                                 rows2.at[b, pl.ds(r * SUB, SUB)], sem_g)

            if with_hist:
                for r in range(NSUB):
                    pltpu.async_copy(hist.at[idxv2.at[b, r]],
                                     vals2.at[b, pl.ds(r * SUB, SUB)], sem_h)
                for r in range(NSUB):
                    pltpu.make_async_copy(
                        hist.at[idxv2.at[b, r]],
                        vals2.at[b, pl.ds(r * SUB, SUB)], sem_h).wait()

                def inv_body(j, _):
                    v = vals2[b, pl.ds(j * LN, LN)]
                    vals2[b, pl.ds(j * LN, LN)] = 1.0 / v
                    return 0
                lax.fori_loop(0, CH // LN, inv_body, 0)
                pltpu.async_copy(vals2.at[b],
                                 vals_h.at[pl.ds(ebase + k * CH, CH)], sem_v)

            for r in range(NSUB):
                pltpu.make_async_copy(
                    tab_h.at[idxw2.at[b, r]],
                    rows2.at[b, pl.ds(r * SUB, SUB)], sem_g).wait()

            def scale_body(j, _):
                sp = plsc.load_gather(vals2, [bs, jnp.full((LN,), j,
                                                           jnp.int32)])
                rows2[b, j] = rows2[b, j] * sp
                return 0
            lax.fori_loop(0, CH, scale_body, 0)

            for r in range(NSUB):
                pltpu.async_copy(rows2.at[b, pl.ds(r * SUB, SUB)],
                                 acc.at[idxf2.at[b, r]], sem_s, add=True)
            return 0
        lax.fori_loop(0, NCH_L, edge_chunk, 0)
        for b in range(2):
            for r in range(NSUB):
                pltpu.make_async_copy(
                    rows2.at[b, pl.ds(r * SUB, SUB)],
                    acc.at[idxf2.at[b, r]], sem_s).wait()
            if with_hist:
                pltpu.make_async_copy(
                    vals2.at[b], vals_h.at[pl.ds(0, CH)], sem_v).wait()

        plsc.subcore_barrier()

        pltpu.sync_copy(acc.at[pl.ds(sid * HR_T, HR_T)],
                        part_h.at[cid].at[pl.ds(sid * HR_T, HR_T)])

    kern = pl.kernel(body, out_type=out_type, mesh=mesh,
                     scratch_types=scratch_types,
                     compiler_params=pltpu.CompilerParams(
                         use_tc_tiling_on_sc=False,
                         needs_layout_passes=False))
    return kern


def _tc_hw2(n_real, r, emb, ncls, blk):
    grid = n_real // blk

    def body(hp_ref, b1_ref, w2_ref, out_ref):
        h = jax.nn.relu(hp_ref[0] + hp_ref[1] + b1_ref[0][None, :])
        for p in range(r):
            out_ref[p] = jnp.dot(h, w2_ref[p],
                                 preferred_element_type=jnp.float32)

    return pl.pallas_call(
        body,
        grid=(grid,),
        in_specs=[
            pl.BlockSpec((NC, blk, emb), lambda i: (0, i, 0)),
            pl.BlockSpec((1, emb), lambda i: (0, 0)),
            pl.BlockSpec((r, emb, ncls), lambda i: (0, 0, 0)),
        ],
        out_specs=pl.BlockSpec((r, blk, ncls), lambda i: (0, i, 0)),
        out_shape=jax.ShapeDtypeStruct((r, n_real, ncls), jnp.float32),
    )


def _tc_combine(n_real, ncls, blk):
    grid = n_real // blk

    def body(op_ref, b2_ref, out_ref):
        out_ref[...] = op_ref[0] + op_ref[1] + b2_ref[0][None, :]

    return pl.pallas_call(
        body,
        grid=(grid,),
        in_specs=[
            pl.BlockSpec((NC, blk, ncls), lambda i: (0, i, 0)),
            pl.BlockSpec((1, ncls), lambda i: (0, 0)),
        ],
        out_specs=pl.BlockSpec((blk, ncls), lambda i: (i, 0)),
        out_shape=jax.ShapeDtypeStruct((n_real, ncls), jnp.float32),
    )


def _rgcn(triples, weights1, weights2, bias1, bias2, cfg):
    n, r = cfg["n"], cfg["r"]
    emb, ncls = cfg["emb"], cfg["ncls"]

    w1_flat = weights1.reshape(r * n, emb)
    fr = triples[:, 0]
    p = triples[:, 1]
    to = triples[:, 2]

    hpart, vals = _make_edge_pass(cfg, with_hist=True)(fr, p, to, w1_flat)

    hw2 = _tc_hw2(n, r, emb, ncls, cfg["tc_blk"])(
        hpart, bias1.reshape(1, emb), weights2)
    hw2_flat = hw2.reshape(r * n, ncls)

    (opart,) = _make_edge_pass(cfg, with_hist=False)(fr, p, to, vals,
                                                     hw2_flat)

    out = _tc_combine(n, ncls, cfg["tc_blk"])(opart, bias2.reshape(1, ncls))
    return out


_CFG_FULL = dict(
    n=50000, r=8, emb=16, ncls=16,
    ch=400, sub=80, t_edge=50000,
    hbins=409600, hrows=50000,
    zf=3200, zr=125,
    tc_blk=2000,
)


def kernel(triples, weights1, weights2, bias1, bias2):
    return _rgcn(triples, weights1, weights2, bias1, bias2, _CFG_FULL)


# speedup vs baseline: 6.0828x; 1.2951x over previous
import jax
import jax.numpy as jnp
from jax import lax
from jax.experimental import pallas as pl
from jax.experimental.pallas import tpu as pltpu
from jax.experimental.pallas import tpu_sc as plsc

NC, NS, LN = 2, 16, 16


def _make_vals_pass(cfg):
    n = cfg["n"]
    CH = cfg["ch"]
    T_EDGE = cfg["t_edge"]
    T_HIST = T_EDGE * NC
    NCH_L = T_EDGE // CH
    NCH_H = T_HIST // CH
    HBINS = cfg["hbins"]
    HB_T = HBINS // NS
    ZF = cfg["zf"]

    mesh = plsc.VectorSubcoreMesh(
        core_axis_name="c", subcore_axis_name="s", num_cores=NC,
        num_subcores=NS)
    e_tot = NC * NS * T_EDGE
    out_type = [jax.ShapeDtypeStruct((e_tot,), jnp.float32)]
    scratch_types = [
        pltpu.VMEM_SHARED((HBINS,), jnp.float32),
        pltpu.VMEM((ZF,), jnp.float32),
        pltpu.VMEM((CH,), jnp.float32),
        pltpu.VMEM((2, CH), jnp.int32),
        pltpu.VMEM((2, CH), jnp.int32),
        pltpu.VMEM((2, CH), jnp.int32),
        pltpu.VMEM((2, CH), jnp.float32),
        pltpu.SemaphoreType.DMA,
        pltpu.SemaphoreType.DMA,
        pltpu.SemaphoreType.DMA,
    ]

    def body(fr_h, p_h, vals_h, hist, zflat, ones, fr2, p2, idxv2, vals2,
             sem_l, sem_s, sem_v):
        cid = lax.axis_index("c")
        sid = lax.axis_index("s")
        wid = cid * NS + sid

        def zf_body(i, _):
            zflat[pl.ds(i * LN, LN)] = jnp.zeros((LN,), jnp.float32)
            return 0
        lax.fori_loop(0, ZF // LN, zf_body, 0)

        def on_body(i, _):
            ones[pl.ds(i * LN, LN)] = jnp.ones((LN,), jnp.float32)
            return 0
        lax.fori_loop(0, CH // LN, on_body, 0)
        for k in range(HB_T // ZF):
            pltpu.async_copy(
                zflat, hist.at[pl.ds(sid * HB_T + k * ZF, ZF)], sem_s)
        for k in range(HB_T // ZF):
            pltpu.make_async_copy(
                zflat, hist.at[pl.ds(sid * HB_T + k * ZF, ZF)], sem_s).wait()
        plsc.subcore_barrier()

        def load2(k, b, base):
            pltpu.async_copy(fr_h.at[pl.ds(base + k * CH, CH)],
                             fr2.at[b], sem_l)
            pltpu.async_copy(p_h.at[pl.ds(base + k * CH, CH)],
                             p2.at[b], sem_l)

        def wait2(b):
            pltpu.make_async_copy(fr_h.at[pl.ds(0, CH)], fr2.at[b],
                                  sem_l).wait()
            pltpu.make_async_copy(p_h.at[pl.ds(0, CH)], p2.at[b],
                                  sem_l).wait()

        hbase = sid * T_HIST
        load2(0, 0, hbase)

        def hist_chunk(k, _):
            b = lax.rem(k, 2)

            @pl.when(k >= 2)
            def _():
                pltpu.make_async_copy(ones, hist.at[idxv2.at[b]],
                                      sem_s).wait()
            wait2(b)

            @pl.when(k + 1 < NCH_H)
            def _():
                load2(k + 1, 1 - b, hbase)

            def vec_body(j, _):
                fj = fr2[b, pl.ds(j * LN, LN)]
                pj = p2[b, pl.ds(j * LN, LN)]
                idxv2[b, pl.ds(j * LN, LN)] = pj * n + fj
                return 0
            lax.fori_loop(0, CH // LN, vec_body, 0)
            pltpu.async_copy(ones, hist.at[idxv2.at[b]], sem_s, add=True)
            return 0
        lax.fori_loop(0, NCH_H, hist_chunk, 0)
        for b in range(2):
            pltpu.make_async_copy(ones, hist.at[idxv2.at[b]], sem_s).wait()
        plsc.subcore_barrier()

        ebase = wid * T_EDGE
        load2(0, 0, ebase)

        def val_chunk(k, _):
            b = lax.rem(k, 2)

            @pl.when(k >= 2)
            def _():
                pltpu.make_async_copy(vals2.at[b],
                                      vals_h.at[pl.ds(0, CH)], sem_v).wait()
            wait2(b)

            @pl.when(k + 1 < NCH_L)
            def _():
                load2(k + 1, 1 - b, ebase)

            def vec_body(j, _):
                fj = fr2[b, pl.ds(j * LN, LN)]
                pj = p2[b, pl.ds(j * LN, LN)]
                idxv2[b, pl.ds(j * LN, LN)] = pj * n + fj
                return 0
            lax.fori_loop(0, CH // LN, vec_body, 0)
            pltpu.sync_copy(hist.at[idxv2.at[b]], vals2.at[b])

            def inv_body(j, _):
                v = vals2[b, pl.ds(j * LN, LN)]
                vals2[b, pl.ds(j * LN, LN)] = 1.0 / v
                return 0
            lax.fori_loop(0, CH // LN, inv_body, 0)
            pltpu.async_copy(vals2.at[b], vals_h.at[pl.ds(ebase + k * CH, CH)],
                             sem_v)
            return 0
        lax.fori_loop(0, NCH_L, val_chunk, 0)
        for b in range(2):
            pltpu.make_async_copy(vals2.at[b], vals_h.at[pl.ds(0, CH)],
                                  sem_v).wait()

    return pl.kernel(body, out_type=out_type, mesh=mesh,
                     scratch_types=scratch_types,
                     compiler_params=pltpu.CompilerParams(
                         use_tc_tiling_on_sc=False,
                         needs_layout_passes=False))


def _make_layer_pass(cfg):
    n = cfg["n"]
    CH = cfg["ch"]
    NSUB = 5
    SUB = CH // NSUB
    T_EDGE = cfg["t_edge"]
    NCH_L = T_EDGE // CH
    HROWS = cfg["hrows"]
    HR_T = HROWS // NS
    ZR = cfg["zr"]
    emb = cfg["emb"]

    mesh = plsc.VectorSubcoreMesh(
        core_axis_name="c", subcore_axis_name="s", num_cores=NC,
        num_subcores=NS)
    out_type = [jax.ShapeDtypeStruct((NC, HROWS, emb), jnp.float32)]
    scratch_types = [
        pltpu.VMEM_SHARED((HROWS, emb), jnp.float32),
        pltpu.VMEM((ZR, emb), jnp.float32),
        pltpu.VMEM((2, CH), jnp.int32),
        pltpu.VMEM((2, CH), jnp.int32),
        pltpu.VMEM((2, CH), jnp.int32),
        pltpu.VMEM((2, CH), jnp.float32),
        pltpu.VMEM((2, NSUB, SUB), jnp.int32),
        pltpu.VMEM((2, NSUB, SUB), jnp.int32),
        pltpu.VMEM((3, SUB, emb), jnp.float32),
        pltpu.SemaphoreType.DMA,
        pltpu.SemaphoreType.DMA,
        pltpu.SemaphoreType.DMA,
    ]

    def body(fr_h, p_h, to_h, vals_in_h, tab_h, part_h,
             acc, zrows, fr2, p2, to2, vals2, idxw2, idxf2, rows2,
             sem_l, sem_g, sem_s):
        cid = lax.axis_index("c")
        sid = lax.axis_index("s")
        wid = cid * NS + sid

        def zr_body(i, _):
            zrows[i] = jnp.zeros((LN,), jnp.float32)
            return 0
        lax.fori_loop(0, ZR, zr_body, 0)
        for k in range(HR_T // ZR):
            pltpu.async_copy(zrows, acc.at[pl.ds(sid * HR_T + k * ZR, ZR)],
                             sem_s)
        for k in range(HR_T // ZR):
            pltpu.make_async_copy(
                zrows, acc.at[pl.ds(sid * HR_T + k * ZR, ZR)], sem_s).wait()
        plsc.subcore_barrier()

        def load4(k, b, base):
            for s, d in ((fr_h, fr2), (p_h, p2), (to_h, to2),
                         (vals_in_h, vals2)):
                pltpu.async_copy(s.at[pl.ds(base + k * CH, CH)],
                                 d.at[b], sem_l)

        def wait4(b):
            for s, d in ((fr_h, fr2), (p_h, p2), (to_h, to2),
                         (vals_in_h, vals2)):
                pltpu.make_async_copy(s.at[pl.ds(0, CH)], d.at[b],
                                      sem_l).wait()

        ebase = wid * T_EDGE
        load4(0, 0, ebase)

        def edge_chunk(k, _):
            b = lax.rem(k, 2)
            wait4(b)

            @pl.when(k + 1 < NCH_L)
            def _():
                load4(k + 1, 1 - b, ebase)

            def vec_body(j, _):
                s = j // (SUB // LN)
                col = (j % (SUB // LN)) * LN
                fj = fr2[b, pl.ds(j * LN, LN)]
                pj = p2[b, pl.ds(j * LN, LN)]
                tj = to2[b, pl.ds(j * LN, LN)]
                idxw2[b, s, pl.ds(col, LN)] = pj * n + tj
                idxf2[b, s, pl.ds(col, LN)] = fj
                return 0
            lax.fori_loop(0, CH // LN, vec_body, 0)

            @pl.when(k >= 1)
            def _():
                pltpu.make_async_copy(
                    rows2.at[lax.rem(k * NSUB, 3)],
                    acc.at[idxf2.at[1 - b, 2]], sem_s).wait()
            pltpu.async_copy(tab_h.at[idxw2.at[b, 0]],
                             rows2.at[lax.rem(k * NSUB, 3)], sem_g)
            for s in range(NSUB):
                g_mod3 = lax.rem(k * NSUB + s, 3)
                nxt_mod3 = lax.rem(k * NSUB + s + 1, 3)
                if s < NSUB - 1:
                    if s >= 2:
                        pltpu.make_async_copy(
                            rows2.at[nxt_mod3],
                            acc.at[idxf2.at[b, s - 2]], sem_s).wait()
                    else:
                        @pl.when(k >= 1)
                        def _():
                            pltpu.make_async_copy(
                                rows2.at[nxt_mod3],
                                acc.at[idxf2.at[1 - b, s + 3]],
                                sem_s).wait()
                    pltpu.async_copy(tab_h.at[idxw2.at[b, s + 1]],
                                     rows2.at[nxt_mod3], sem_g)
                pltpu.make_async_copy(tab_h.at[idxw2.at[b, s]],
                                      rows2.at[g_mod3], sem_g).wait()

                def scale_body(j, _):
                    sp = plsc.load_gather(
                        vals2, [jnp.full((LN,), b, jnp.int32),
                                jnp.full((LN,), s * SUB + j, jnp.int32)])
                    rows2[g_mod3, j] = rows2[g_mod3, j] * sp
                    return 0
                lax.fori_loop(0, SUB, scale_body, 0)
                pltpu.async_copy(rows2.at[g_mod3], acc.at[idxf2.at[b, s]],
                                 sem_s, add=True)
            return 0
        lax.fori_loop(0, NCH_L, edge_chunk, 0)
        bl = (NCH_L - 1) % 2
        for s in (2, 3, 4):
            g = (NCH_L - 1) * NSUB + s
            pltpu.make_async_copy(rows2.at[g % 3],
                                  acc.at[idxf2.at[bl, s]], sem_s).wait()

        plsc.subcore_barrier()
        pltpu.sync_copy(acc.at[pl.ds(sid * HR_T, HR_T)],
                        part_h.at[cid].at[pl.ds(sid * HR_T, HR_T)])

    return pl.kernel(body, out_type=out_type, mesh=mesh,
                     scratch_types=scratch_types,
                     compiler_params=pltpu.CompilerParams(
                         use_tc_tiling_on_sc=False,
                         needs_layout_passes=False))


def _tc_hw2(n_real, r, emb, ncls, blk):
    grid = n_real // blk

    def body(hp_ref, b1_ref, w2_ref, out_ref):
        h = jax.nn.relu(hp_ref[0] + hp_ref[1] + b1_ref[0][None, :])
        for p in range(r):
            out_ref[p] = jnp.dot(h, w2_ref[p],
                                 preferred_element_type=jnp.float32)

    return pl.pallas_call(
        body,
        grid=(grid,),
        in_specs=[
            pl.BlockSpec((NC, blk, emb), lambda i: (0, i, 0)),
            pl.BlockSpec((1, emb), lambda i: (0, 0)),
            pl.BlockSpec((r, emb, ncls), lambda i: (0, 0, 0)),
        ],
        out_specs=pl.BlockSpec((r, blk, ncls), lambda i: (0, i, 0)),
        out_shape=jax.ShapeDtypeStruct((r, n_real, ncls), jnp.float32),
    )


def _tc_combine(n_real, ncls, blk):
    grid = n_real // blk

    def body(op_ref, b2_ref, out_ref):
        out_ref[...] = op_ref[0] + op_ref[1] + b2_ref[0][None, :]

    return pl.pallas_call(
        body,
        grid=(grid,),
        in_specs=[
            pl.BlockSpec((NC, blk, ncls), lambda i: (0, i, 0)),
            pl.BlockSpec((1, ncls), lambda i: (0, 0)),
        ],
        out_specs=pl.BlockSpec((blk, ncls), lambda i: (i, 0)),
        out_shape=jax.ShapeDtypeStruct((n_real, ncls), jnp.float32),
    )


def _rgcn(triples, weights1, weights2, bias1, bias2, cfg):
    n, r = cfg["n"], cfg["r"]
    emb, ncls = cfg["emb"], cfg["ncls"]

    w1_flat = weights1.reshape(r * n, emb)
    fr = triples[:, 0]
    p = triples[:, 1]
    to = triples[:, 2]

    (vals,) = _make_vals_pass(cfg)(fr, p)
    layer = _make_layer_pass(cfg)

    (hpart,) = layer(fr, p, to, vals, w1_flat)

    hw2 = _tc_hw2(n, r, emb, ncls, cfg["tc_blk"])(
        hpart, bias1.reshape(1, emb), weights2)
    hw2_flat = hw2.reshape(r * n, ncls)

    (opart,) = layer(fr, p, to, vals, hw2_flat)

    out = _tc_combine(n, ncls, cfg["tc_blk"])(opart, bias2.reshape(1, ncls))
    return out


_CFG_FULL = dict(
    n=50000, r=8, emb=16, ncls=16,
    ch=2000, t_edge=50000,
    hbins=409600, hrows=50000,
    zf=3200, zr=125,
    tc_blk=2000,
)


def kernel(triples, weights1, weights2, bias1, bias2):
    return _rgcn(triples, weights1, weights2, bias1, bias2, _CFG_FULL)


# speedup vs baseline: 7.1295x; 1.1721x over previous
import jax
import jax.numpy as jnp
from jax import lax
from jax.experimental import pallas as pl
from jax.experimental.pallas import tpu as pltpu
from jax.experimental.pallas import tpu_sc as plsc

NC, NS, LN = 2, 16, 16


def _make_vals_pass(cfg):
    n = cfg["n"]
    CH = cfg["ch"]
    T_EDGE = cfg["t_edge"]
    T_HIST = T_EDGE * NC
    NCH_L = T_EDGE // CH
    NCH_H = T_HIST // CH
    HBINS = cfg["hbins"]
    HB_T = HBINS // NS
    ZF = cfg["zf"]

    mesh = plsc.VectorSubcoreMesh(
        core_axis_name="c", subcore_axis_name="s", num_cores=NC,
        num_subcores=NS)
    e_tot = NC * NS * T_EDGE
    out_type = [jax.ShapeDtypeStruct((e_tot,), jnp.float32)]
    scratch_types = [
        pltpu.VMEM_SHARED((HBINS,), jnp.float32),
        pltpu.VMEM((ZF,), jnp.float32),
        pltpu.VMEM((CH,), jnp.float32),
        pltpu.VMEM((2, CH), jnp.int32),
        pltpu.VMEM((2, CH), jnp.int32),
        pltpu.VMEM((2, CH), jnp.int32),
        pltpu.VMEM((2, CH), jnp.float32),
        pltpu.SemaphoreType.DMA,
        pltpu.SemaphoreType.DMA,
        pltpu.SemaphoreType.DMA,
    ]

    def body(fr_h, p_h, vals_h, hist, zflat, ones, fr2, p2, idxv2, vals2,
             sem_l, sem_s, sem_v):
        cid = lax.axis_index("c")
        sid = lax.axis_index("s")
        wid = cid * NS + sid

        def zf_body(i, _):
            zflat[pl.ds(i * LN, LN)] = jnp.zeros((LN,), jnp.float32)
            return 0
        lax.fori_loop(0, ZF // LN, zf_body, 0)

        def on_body(i, _):
            ones[pl.ds(i * LN, LN)] = jnp.ones((LN,), jnp.float32)
            return 0
        lax.fori_loop(0, CH // LN, on_body, 0)
        for k in range(HB_T // ZF):
            pltpu.async_copy(
                zflat, hist.at[pl.ds(sid * HB_T + k * ZF, ZF)], sem_s)
        for k in range(HB_T // ZF):
            pltpu.make_async_copy(
                zflat, hist.at[pl.ds(sid * HB_T + k * ZF, ZF)], sem_s).wait()
        plsc.subcore_barrier()

        def load2(k, b, base):
            pltpu.async_copy(fr_h.at[pl.ds(base + k * CH, CH)],
                             fr2.at[b], sem_l)
            pltpu.async_copy(p_h.at[pl.ds(base + k * CH, CH)],
                             p2.at[b], sem_l)

        def wait2(b):
            pltpu.make_async_copy(fr_h.at[pl.ds(0, CH)], fr2.at[b],
                                  sem_l).wait()
            pltpu.make_async_copy(p_h.at[pl.ds(0, CH)], p2.at[b],
                                  sem_l).wait()

        hbase = sid * T_HIST
        load2(0, 0, hbase)

        def hist_chunk(k, _):
            b = lax.rem(k, 2)

            @pl.when(k >= 2)
            def _():
                pltpu.make_async_copy(ones, hist.at[idxv2.at[b]],
                                      sem_s).wait()
            wait2(b)

            @pl.when(k + 1 < NCH_H)
            def _():
                load2(k + 1, 1 - b, hbase)

            def vec_body(j, _):
                fj = fr2[b, pl.ds(j * LN, LN)]
                pj = p2[b, pl.ds(j * LN, LN)]
                idxv2[b, pl.ds(j * LN, LN)] = pj * n + fj
                return 0
            lax.fori_loop(0, CH // LN, vec_body, 0)
            pltpu.async_copy(ones, hist.at[idxv2.at[b]], sem_s, add=True)
            return 0
        lax.fori_loop(0, NCH_H, hist_chunk, 0)
        for b in range(2):
            pltpu.make_async_copy(ones, hist.at[idxv2.at[b]], sem_s).wait()
        plsc.subcore_barrier()

        ebase = wid * T_EDGE
        load2(0, 0, ebase)

        def val_chunk(k, _):
            b = lax.rem(k, 2)

            @pl.when(k >= 2)
            def _():
                pltpu.make_async_copy(vals2.at[b],
                                      vals_h.at[pl.ds(0, CH)], sem_v).wait()
            wait2(b)

            @pl.when(k + 1 < NCH_L)
            def _():
                load2(k + 1, 1 - b, ebase)

            def vec_body(j, _):
                fj = fr2[b, pl.ds(j * LN, LN)]
                pj = p2[b, pl.ds(j * LN, LN)]
                idxv2[b, pl.ds(j * LN, LN)] = pj * n + fj
                return 0
            lax.fori_loop(0, CH // LN, vec_body, 0)
            pltpu.sync_copy(hist.at[idxv2.at[b]], vals2.at[b])

            def inv_body(j, _):
                v = vals2[b, pl.ds(j * LN, LN)]
                vals2[b, pl.ds(j * LN, LN)] = 1.0 / v
                return 0
            lax.fori_loop(0, CH // LN, inv_body, 0)
            pltpu.async_copy(vals2.at[b], vals_h.at[pl.ds(ebase + k * CH, CH)],
                             sem_v)
            return 0
        lax.fori_loop(0, NCH_L, val_chunk, 0)
        for b in range(2):
            pltpu.make_async_copy(vals2.at[b], vals_h.at[pl.ds(0, CH)],
                                  sem_v).wait()

    return pl.kernel(body, out_type=out_type, mesh=mesh,
                     scratch_types=scratch_types,
                     compiler_params=pltpu.CompilerParams(
                         use_tc_tiling_on_sc=False,
                         needs_layout_passes=False))


def _make_layer_pass(cfg, cp, ct):
    n = cfg["n"]
    CH = cfg["ch"]
    NSUB = 5
    SUB = CH // NSUB
    T_EDGE = cfg["t_edge"]
    NCH_L = T_EDGE // CH
    HROWS = cfg["hrows"]
    HR_T = HROWS // NS
    ZR = cfg["zr"]
    emb = cfg["emb"]

    mesh = plsc.VectorSubcoreMesh(
        core_axis_name="c", subcore_axis_name="s", num_cores=NC,
        num_subcores=NS)
    out_type = [jax.ShapeDtypeStruct((NC, HROWS, emb), jnp.float32)]
    scratch_types = [
        pltpu.VMEM_SHARED((HROWS, emb), jnp.float32),
        pltpu.VMEM((ZR, emb), jnp.float32),
        pltpu.VMEM((2, CH), jnp.int32),
        pltpu.VMEM((2, CH), jnp.int32),
        pltpu.VMEM((2, CH), jnp.int32),
        pltpu.VMEM((2, CH), jnp.float32),
        pltpu.VMEM((2, NSUB, SUB), jnp.int32),
        pltpu.VMEM((2, NSUB, SUB), jnp.int32),
        pltpu.VMEM((3, SUB, emb), jnp.float32),
        pltpu.SemaphoreType.DMA,
        pltpu.SemaphoreType.DMA,
        pltpu.SemaphoreType.DMA,
    ]

    def body(fr_h, p_h, to_h, vals_in_h, tab_h, part_h,
             acc, zrows, fr2, p2, to2, vals2, idxw2, idxf2, rows2,
             sem_l, sem_g, sem_s):
        cid = lax.axis_index("c")
        sid = lax.axis_index("s")
        wid = cid * NS + sid

        def zr_body(i, _):
            zrows[i] = jnp.zeros((LN,), jnp.float32)
            return 0
        lax.fori_loop(0, ZR, zr_body, 0)
        for k in range(HR_T // ZR):
            pltpu.async_copy(zrows, acc.at[pl.ds(sid * HR_T + k * ZR, ZR)],
                             sem_s)
        for k in range(HR_T // ZR):
            pltpu.make_async_copy(
                zrows, acc.at[pl.ds(sid * HR_T + k * ZR, ZR)], sem_s).wait()
        plsc.subcore_barrier()

        def load4(k, b, base):
            for s, d in ((fr_h, fr2), (p_h, p2), (to_h, to2),
                         (vals_in_h, vals2)):
                pltpu.async_copy(s.at[pl.ds(base + k * CH, CH)],
                                 d.at[b], sem_l)

        def wait4(b):
            for s, d in ((fr_h, fr2), (p_h, p2), (to_h, to2),
                         (vals_in_h, vals2)):
                pltpu.make_async_copy(s.at[pl.ds(0, CH)], d.at[b],
                                      sem_l).wait()

        ebase = wid * T_EDGE
        load4(0, 0, ebase)

        def edge_chunk(k, _):
            b = lax.rem(k, 2)
            wait4(b)

            @pl.when(k + 1 < NCH_L)
            def _():
                load4(k + 1, 1 - b, ebase)

            def vec_body(j, _):
                s = j // (SUB // LN)
                col = (j % (SUB // LN)) * LN
                fj = fr2[b, pl.ds(j * LN, LN)]
                pj = p2[b, pl.ds(j * LN, LN)]
                tj = to2[b, pl.ds(j * LN, LN)]
                idxw2[b, s, pl.ds(col, LN)] = pj * cp + tj * ct
                idxf2[b, s, pl.ds(col, LN)] = fj
                return 0
            lax.fori_loop(0, CH // LN, vec_body, 0)

            @pl.when(k >= 1)
            def _():
                pltpu.make_async_copy(
                    rows2.at[lax.rem(k * NSUB, 3)],
                    acc.at[idxf2.at[1 - b, 2]], sem_s).wait()
            pltpu.async_copy(tab_h.at[idxw2.at[b, 0]],
                             rows2.at[lax.rem(k * NSUB, 3)], sem_g)
            for s in range(NSUB):
                g_mod3 = lax.rem(k * NSUB + s, 3)
                nxt_mod3 = lax.rem(k * NSUB + s + 1, 3)
                if s < NSUB - 1:
                    if s >= 2:
                        pltpu.make_async_copy(
                            rows2.at[nxt_mod3],
                            acc.at[idxf2.at[b, s - 2]], sem_s).wait()
                    else:
                        @pl.when(k >= 1)
                        def _():
                            pltpu.make_async_copy(
                                rows2.at[nxt_mod3],
                                acc.at[idxf2.at[1 - b, s + 3]],
                                sem_s).wait()
                    pltpu.async_copy(tab_h.at[idxw2.at[b, s + 1]],
                                     rows2.at[nxt_mod3], sem_g)
                pltpu.make_async_copy(tab_h.at[idxw2.at[b, s]],
                                      rows2.at[g_mod3], sem_g).wait()

                def scale_body(j, _):
                    sp = plsc.load_gather(
                        vals2, [jnp.full((LN,), b, jnp.int32),
                                jnp.full((LN,), s * SUB + j, jnp.int32)])
                    rows2[g_mod3, j] = rows2[g_mod3, j] * sp
                    return 0
                lax.fori_loop(0, SUB, scale_body, 0)
                pltpu.async_copy(rows2.at[g_mod3], acc.at[idxf2.at[b, s]],
                                 sem_s, add=True)
            return 0
        lax.fori_loop(0, NCH_L, edge_chunk, 0)
        bl = (NCH_L - 1) % 2
        for s in (2, 3, 4):
            g = (NCH_L - 1) * NSUB + s
            pltpu.make_async_copy(rows2.at[g % 3],
                                  acc.at[idxf2.at[bl, s]], sem_s).wait()

        plsc.subcore_barrier()
        pltpu.sync_copy(acc.at[pl.ds(sid * HR_T, HR_T)],
                        part_h.at[cid].at[pl.ds(sid * HR_T, HR_T)])

    return pl.kernel(body, out_type=out_type, mesh=mesh,
                     scratch_types=scratch_types,
                     compiler_params=pltpu.CompilerParams(
                         use_tc_tiling_on_sc=False,
                         needs_layout_passes=False))


def _tc_hw2(n_real, r, emb, ncls, blk):
    grid = n_real // blk

    def body(hp_ref, b1_ref, w2_ref, out_ref):
        h = jax.nn.relu(hp_ref[0] + hp_ref[1] + b1_ref[0][None, :])
        out_ref[...] = jnp.dot(h, w2_ref[...],
                               preferred_element_type=jnp.float32)

    return pl.pallas_call(
        body,
        grid=(grid,),
        in_specs=[
            pl.BlockSpec((NC, blk, emb), lambda i: (0, i, 0)),
            pl.BlockSpec((1, emb), lambda i: (0, 0)),
            pl.BlockSpec((emb, r * ncls), lambda i: (0, 0)),
        ],
        out_specs=pl.BlockSpec((blk, r * ncls), lambda i: (i, 0)),
        out_shape=jax.ShapeDtypeStruct((n_real, r * ncls), jnp.float32),
    )


def _tc_combine(n_real, ncls, blk):
    grid = n_real // blk

    def body(op_ref, b2_ref, out_ref):
        out_ref[...] = op_ref[0] + op_ref[1] + b2_ref[0][None, :]

    return pl.pallas_call(
        body,
        grid=(grid,),
        in_specs=[
            pl.BlockSpec((NC, blk, ncls), lambda i: (0, i, 0)),
            pl.BlockSpec((1, ncls), lambda i: (0, 0)),
        ],
        out_specs=pl.BlockSpec((blk, ncls), lambda i: (i, 0)),
        out_shape=jax.ShapeDtypeStruct((n_real, ncls), jnp.float32),
    )


def _rgcn(triples, weights1, weights2, bias1, bias2, cfg):
    n, r = cfg["n"], cfg["r"]
    emb, ncls = cfg["emb"], cfg["ncls"]

    w1_flat = weights1.reshape(r * n, emb)
    fr = triples[:, 0]
    p = triples[:, 1]
    to = triples[:, 2]

    (vals,) = _make_vals_pass(cfg)(fr, p)

    (hpart,) = _make_layer_pass(cfg, n, 1)(fr, p, to, vals, w1_flat)

    w2cat = jnp.transpose(weights2, (1, 0, 2)).reshape(emb, r * ncls)
    hw2 = _tc_hw2(n, r, emb, ncls, cfg["tc_blk"])(
        hpart, bias1.reshape(1, emb), w2cat)
    hw2_flat = hw2.reshape(n * r, ncls)

    (opart,) = _make_layer_pass(cfg, 1, r)(fr, p, to, vals, hw2_flat)

    out = _tc_combine(n, ncls, cfg["tc_blk"])(opart, bias2.reshape(1, ncls))
    return out


_CFG_FULL = dict(
    n=50000, r=8, emb=16, ncls=16,
    ch=2000, t_edge=50000,
    hbins=409600, hrows=50000,
    zf=3200, zr=125,
    tc_blk=2000,
)


def kernel(triples, weights1, weights2, bias1, bias2):
    return _rgcn(triples, weights1, weights2, bias1, bias2, _CFG_FULL)


# speedup vs baseline: 10.4417x; 1.4646x over previous
import jax
import jax.numpy as jnp
from jax import lax
from jax.experimental import pallas as pl
from jax.experimental.pallas import tpu as pltpu
from jax.experimental.pallas import tpu_sc as plsc

NC, NS, LN = 2, 16, 16


def _make_vals_pass(cfg):
    n = cfg["n"]
    CH = cfg["ch"]
    T_EDGE = cfg["t_edge"]
    T_HIST = T_EDGE * NC
    NCH_L = T_EDGE // CH
    NCH_H = T_HIST // CH
    HBINS = cfg["hbins"]
    HB_T = HBINS // NS
    ZF = cfg["zf"]

    mesh = plsc.VectorSubcoreMesh(
        core_axis_name="c", subcore_axis_name="s", num_cores=NC,
        num_subcores=NS)
    e_tot = NC * NS * T_EDGE
    out_type = [jax.ShapeDtypeStruct((e_tot,), jnp.float32)]
    scratch_types = [
        pltpu.VMEM_SHARED((HBINS,), jnp.float32),
        pltpu.VMEM((ZF,), jnp.float32),
        pltpu.VMEM((CH,), jnp.float32),
        pltpu.VMEM((2, CH), jnp.int32),
        pltpu.VMEM((2, CH), jnp.int32),
        pltpu.VMEM((2, CH), jnp.int32),
        pltpu.VMEM((2, CH), jnp.float32),
        pltpu.SemaphoreType.DMA,
        pltpu.SemaphoreType.DMA,
        pltpu.SemaphoreType.DMA,
    ]

    def body(fr_h, p_h, vals_h, hist, zflat, ones, fr2, p2, idxv2, vals2,
             sem_l, sem_s, sem_v):
        cid = lax.axis_index("c")
        sid = lax.axis_index("s")
        wid = cid * NS + sid

        def zf_body(i, _):
            zflat[pl.ds(i * LN, LN)] = jnp.zeros((LN,), jnp.float32)
            return 0
        lax.fori_loop(0, ZF // LN, zf_body, 0)

        def on_body(i, _):
            ones[pl.ds(i * LN, LN)] = jnp.ones((LN,), jnp.float32)
            return 0
        lax.fori_loop(0, CH // LN, on_body, 0)
        for k in range(HB_T // ZF):
            pltpu.async_copy(
                zflat, hist.at[pl.ds(sid * HB_T + k * ZF, ZF)], sem_s)
        for k in range(HB_T // ZF):
            pltpu.make_async_copy(
                zflat, hist.at[pl.ds(sid * HB_T + k * ZF, ZF)], sem_s).wait()
        plsc.subcore_barrier()

        def load2(k, b, base):
            pltpu.async_copy(fr_h.at[pl.ds(base + k * CH, CH)],
                             fr2.at[b], sem_l)
            pltpu.async_copy(p_h.at[pl.ds(base + k * CH, CH)],
                             p2.at[b], sem_l)

        def wait2(b):
            pltpu.make_async_copy(fr_h.at[pl.ds(0, CH)], fr2.at[b],
                                  sem_l).wait()
            pltpu.make_async_copy(p_h.at[pl.ds(0, CH)], p2.at[b],
                                  sem_l).wait()

        hbase = sid * T_HIST
        load2(0, 0, hbase)

        def hist_chunk(k, _):
            b = lax.rem(k, 2)

            @pl.when(k >= 2)
            def _():
                pltpu.make_async_copy(ones, hist.at[idxv2.at[b]],
                                      sem_s).wait()
            wait2(b)

            @pl.when(k + 1 < NCH_H)
            def _():
                load2(k + 1, 1 - b, hbase)

            def vec_body(j, _):
                fj = fr2[b, pl.ds(j * LN, LN)]
                pj = p2[b, pl.ds(j * LN, LN)]
                idxv2[b, pl.ds(j * LN, LN)] = pj * n + fj
                return 0
            lax.fori_loop(0, CH // LN, vec_body, 0)
            pltpu.async_copy(ones, hist.at[idxv2.at[b]], sem_s, add=True)
            return 0
        lax.fori_loop(0, NCH_H, hist_chunk, 0)
        for b in range(2):
            pltpu.make_async_copy(ones, hist.at[idxv2.at[b]], sem_s).wait()
        plsc.subcore_barrier()

        ebase = wid * T_EDGE
        load2(0, 0, ebase)

        def val_chunk(k, _):
            b = lax.rem(k, 2)

            @pl.when(k >= 2)
            def _():
                pltpu.make_async_copy(vals2.at[b],
                                      vals_h.at[pl.ds(0, CH)], sem_v).wait()
            wait2(b)

            @pl.when(k + 1 < NCH_L)
            def _():
                load2(k + 1, 1 - b, ebase)

            def vec_body(j, _):
                fj = fr2[b, pl.ds(j * LN, LN)]
                pj = p2[b, pl.ds(j * LN, LN)]
                idxv2[b, pl.ds(j * LN, LN)] = pj * n + fj
                return 0
            lax.fori_loop(0, CH // LN, vec_body, 0)
            pltpu.sync_copy(hist.at[idxv2.at[b]], vals2.at[b])

            def inv_body(j, _):
                v = vals2[b, pl.ds(j * LN, LN)]
                vals2[b, pl.ds(j * LN, LN)] = 1.0 / v
                return 0
            lax.fori_loop(0, CH // LN, inv_body, 0)
            pltpu.async_copy(vals2.at[b], vals_h.at[pl.ds(ebase + k * CH, CH)],
                             sem_v)
            return 0
        lax.fori_loop(0, NCH_L, val_chunk, 0)
        for b in range(2):
            pltpu.make_async_copy(vals2.at[b], vals_h.at[pl.ds(0, CH)],
                                  sem_v).wait()

    return pl.kernel(body, out_type=out_type, mesh=mesh,
                     scratch_types=scratch_types,
                     compiler_params=pltpu.CompilerParams(
                         use_tc_tiling_on_sc=False,
                         needs_layout_passes=False))


def _make_layer_pass(cfg, cp, ct):
    n = cfg["n"]
    CH = cfg["ch"]
    NSUB = 5
    SUB = CH // NSUB
    T_EDGE = cfg["t_edge"]
    NCH_L = T_EDGE // CH
    HROWS = cfg["hrows"]
    HR_T = HROWS // NS
    ZR = cfg["zr"]
    emb = cfg["emb"]

    mesh = plsc.VectorSubcoreMesh(
        core_axis_name="c", subcore_axis_name="s", num_cores=NC,
        num_subcores=NS)
    out_type = [jax.ShapeDtypeStruct((NC, HROWS, emb), jnp.float32)]
    scratch_types = [
        pltpu.VMEM_SHARED((HROWS, emb), jnp.float32),
        pltpu.VMEM((ZR, emb), jnp.float32),
        pltpu.VMEM((2, CH), jnp.int32),
        pltpu.VMEM((2, CH), jnp.int32),
        pltpu.VMEM((2, CH), jnp.int32),
        pltpu.VMEM((2, CH), jnp.float32),
        pltpu.VMEM((2, NSUB, SUB), jnp.int32),
        pltpu.VMEM((2, NSUB, SUB), jnp.int32),
        pltpu.VMEM((3, SUB, emb), jnp.float32),
        pltpu.SemaphoreType.DMA,
        pltpu.SemaphoreType.DMA,
        pltpu.SemaphoreType.DMA,
    ]

    def body(fr_h, p_h, to_h, vals_in_h, tab_h, part_h,
             acc, zrows, fr2, p2, to2, vals2, idxw2, idxf2, rows2,
             sem_l, sem_g, sem_s):
        cid = lax.axis_index("c")
        sid = lax.axis_index("s")
        wid = cid * NS + sid

        def zr_body(i, _):
            zrows[i] = jnp.zeros((LN,), jnp.float32)
            return 0
        lax.fori_loop(0, ZR, zr_body, 0)
        for k in range(HR_T // ZR):
            pltpu.async_copy(zrows, acc.at[pl.ds(sid * HR_T + k * ZR, ZR)],
                             sem_s)
        for k in range(HR_T // ZR):
            pltpu.make_async_copy(
                zrows, acc.at[pl.ds(sid * HR_T + k * ZR, ZR)], sem_s).wait()
        plsc.subcore_barrier()

        def load4(k, b, base):
            for s, d in ((fr_h, fr2), (p_h, p2), (to_h, to2),
                         (vals_in_h, vals2)):
                pltpu.async_copy(s.at[pl.ds(base + k * CH, CH)],
                                 d.at[b], sem_l)

        def wait4(b):
            for s, d in ((fr_h, fr2), (p_h, p2), (to_h, to2),
                         (vals_in_h, vals2)):
                pltpu.make_async_copy(s.at[pl.ds(0, CH)], d.at[b],
                                      sem_l).wait()

        ebase = wid * T_EDGE
        load4(0, 0, ebase)

        def edge_chunk(k, _):
            b = lax.rem(k, 2)
            wait4(b)

            @pl.when(k + 1 < NCH_L)
            def _():
                load4(k + 1, 1 - b, ebase)

            def vec_body(j, _):
                s = j // (SUB // LN)
                col = (j % (SUB // LN)) * LN
                fj = fr2[b, pl.ds(j * LN, LN)]
                pj = p2[b, pl.ds(j * LN, LN)]
                tj = to2[b, pl.ds(j * LN, LN)]
                idxw2[b, s, pl.ds(col, LN)] = pj * cp + tj * ct
                idxf2[b, s, pl.ds(col, LN)] = fj
                return 0
            lax.fori_loop(0, CH // LN, vec_body, 0)

            @pl.when(k >= 1)
            def _():
                pltpu.make_async_copy(
                    rows2.at[lax.rem(k * NSUB, 3)],
                    acc.at[idxf2.at[1 - b, 2]], sem_s).wait()
            pltpu.async_copy(tab_h.at[idxw2.at[b, 0]],
                             rows2.at[lax.rem(k * NSUB, 3)], sem_g)
            for s in range(NSUB):
                g_mod3 = lax.rem(k * NSUB + s, 3)
                nxt_mod3 = lax.rem(k * NSUB + s + 1, 3)
                if s < NSUB - 1:
                    if s >= 2:
                        pltpu.make_async_copy(
                            rows2.at[nxt_mod3],
                            acc.at[idxf2.at[b, s - 2]], sem_s).wait()
                    else:
                        @pl.when(k >= 1)
                        def _():
                            pltpu.make_async_copy(
                                rows2.at[nxt_mod3],
                                acc.at[idxf2.at[1 - b, s + 3]],
                                sem_s).wait()
                    pltpu.async_copy(tab_h.at[idxw2.at[b, s + 1]],
                                     rows2.at[nxt_mod3], sem_g)
                pltpu.make_async_copy(tab_h.at[idxw2.at[b, s]],
                                      rows2.at[g_mod3], sem_g).wait()

                @plsc.parallel_loop(0, SUB, step=1, unroll=8)
                def _(j):
                    sp = plsc.load_gather(
                        vals2, [jnp.full((LN,), b, jnp.int32),
                                jnp.full((LN,), s * SUB + j, jnp.int32)])
                    rows2[g_mod3, j] = rows2[g_mod3, j] * sp
                pltpu.async_copy(rows2.at[g_mod3], acc.at[idxf2.at[b, s]],
                                 sem_s, add=True)
            return 0
        lax.fori_loop(0, NCH_L, edge_chunk, 0)
        bl = (NCH_L - 1) % 2
        for s in (2, 3, 4):
            g = (NCH_L - 1) * NSUB + s
            pltpu.make_async_copy(rows2.at[g % 3],
                                  acc.at[idxf2.at[bl, s]], sem_s).wait()

        plsc.subcore_barrier()
        pltpu.sync_copy(acc.at[pl.ds(sid * HR_T, HR_T)],
                        part_h.at[cid].at[pl.ds(sid * HR_T, HR_T)])

    return pl.kernel(body, out_type=out_type, mesh=mesh,
                     scratch_types=scratch_types,
                     compiler_params=pltpu.CompilerParams(
                         use_tc_tiling_on_sc=False,
                         needs_layout_passes=False))


def _tc_hw2(n_real, r, emb, ncls, blk):
    grid = n_real // blk

    def body(hp_ref, b1_ref, w2_ref, out_ref):
        h = jax.nn.relu(hp_ref[0] + hp_ref[1] + b1_ref[0][None, :])
        out_ref[...] = jnp.dot(h, w2_ref[...],
                               preferred_element_type=jnp.float32)

    return pl.pallas_call(
        body,
        grid=(grid,),
        in_specs=[
            pl.BlockSpec((NC, blk, emb), lambda i: (0, i, 0)),
            pl.BlockSpec((1, emb), lambda i: (0, 0)),
            pl.BlockSpec((emb, r * ncls), lambda i: (0, 0)),
        ],
        out_specs=pl.BlockSpec((blk, r * ncls), lambda i: (i, 0)),
        out_shape=jax.ShapeDtypeStruct((n_real, r * ncls), jnp.float32),
    )


def _tc_combine(n_real, ncls, blk):
    grid = n_real // blk

    def body(op_ref, b2_ref, out_ref):
        out_ref[...] = op_ref[0] + op_ref[1] + b2_ref[0][None, :]

    return pl.pallas_call(
        body,
        grid=(grid,),
        in_specs=[
            pl.BlockSpec((NC, blk, ncls), lambda i: (0, i, 0)),
            pl.BlockSpec((1, ncls), lambda i: (0, 0)),
        ],
        out_specs=pl.BlockSpec((blk, ncls), lambda i: (i, 0)),
        out_shape=jax.ShapeDtypeStruct((n_real, ncls), jnp.float32),
    )


def _rgcn(triples, weights1, weights2, bias1, bias2, cfg):
    n, r = cfg["n"], cfg["r"]
    emb, ncls = cfg["emb"], cfg["ncls"]

    w1_flat = weights1.reshape(r * n, emb)
    fr = triples[:, 0]
    p = triples[:, 1]
    to = triples[:, 2]

    (vals,) = _make_vals_pass(cfg)(fr, p)

    (hpart,) = _make_layer_pass(cfg, n, 1)(fr, p, to, vals, w1_flat)

    w2cat = jnp.transpose(weights2, (1, 0, 2)).reshape(emb, r * ncls)
    hw2 = _tc_hw2(n, r, emb, ncls, cfg["tc_blk"])(
        hpart, bias1.reshape(1, emb), w2cat)
    hw2_flat = hw2.reshape(n * r, ncls)

    (opart,) = _make_layer_pass(cfg, 1, r)(fr, p, to, vals, hw2_flat)

    out = _tc_combine(n, ncls, cfg["tc_blk"])(opart, bias2.reshape(1, ncls))
    return out


_CFG_FULL = dict(
    n=50000, r=8, emb=16, ncls=16,
    ch=2000, t_edge=50000,
    hbins=409600, hrows=50000,
    zf=3200, zr=125,
    tc_blk=2000,
)


def kernel(triples, weights1, weights2, bias1, bias2):
    return _rgcn(triples, weights1, weights2, bias1, bias2, _CFG_FULL)


# speedup vs baseline: 10.7198x; 1.0266x over previous
import jax
import jax.numpy as jnp
from jax import lax
from jax.experimental import pallas as pl
from jax.experimental.pallas import tpu as pltpu
from jax.experimental.pallas import tpu_sc as plsc

NC, NS, LN = 2, 16, 16


def _make_vals_pass(cfg):
    n = cfg["n"]
    CH = cfg["ch"]
    T_EDGE = cfg["t_edge"]
    T_HIST = T_EDGE * NC
    NCH_L = T_EDGE // CH
    NCH_H = T_HIST // CH
    HBINS = cfg["hbins"]
    HB_T = HBINS // NS
    ZF = cfg["zf"]

    mesh = plsc.VectorSubcoreMesh(
        core_axis_name="c", subcore_axis_name="s", num_cores=NC,
        num_subcores=NS)
    e_tot = NC * NS * T_EDGE
    out_type = [jax.ShapeDtypeStruct((e_tot,), jnp.float32)]
    scratch_types = [
        pltpu.VMEM_SHARED((HBINS,), jnp.float32),
        pltpu.VMEM((ZF,), jnp.float32),
        pltpu.VMEM((CH,), jnp.float32),
        pltpu.VMEM((2, CH), jnp.int32),
        pltpu.VMEM((2, CH), jnp.int32),
        pltpu.VMEM((2, CH), jnp.int32),
        pltpu.VMEM((2, CH), jnp.float32),
        pltpu.SemaphoreType.DMA,
        pltpu.SemaphoreType.DMA,
        pltpu.SemaphoreType.DMA,
    ]

    def body(fr_h, p_h, vals_h, hist, zflat, ones, fr2, p2, idxv2, vals2,
             sem_l, sem_s, sem_v):
        cid = lax.axis_index("c")
        sid = lax.axis_index("s")
        wid = cid * NS + sid

        def zf_body(i, _):
            zflat[pl.ds(i * LN, LN)] = jnp.zeros((LN,), jnp.float32)
            return 0
        lax.fori_loop(0, ZF // LN, zf_body, 0)

        def on_body(i, _):
            ones[pl.ds(i * LN, LN)] = jnp.ones((LN,), jnp.float32)
            return 0
        lax.fori_loop(0, CH // LN, on_body, 0)
        for k in range(HB_T // ZF):
            pltpu.async_copy(
                zflat, hist.at[pl.ds(sid * HB_T + k * ZF, ZF)], sem_s)
        for k in range(HB_T // ZF):
            pltpu.make_async_copy(
                zflat, hist.at[pl.ds(sid * HB_T + k * ZF, ZF)], sem_s).wait()
        plsc.subcore_barrier()

        def load2(k, b, base):
            pltpu.async_copy(fr_h.at[pl.ds(base + k * CH, CH)],
                             fr2.at[b], sem_l)
            pltpu.async_copy(p_h.at[pl.ds(base + k * CH, CH)],
                             p2.at[b], sem_l)

        def wait2(b):
            pltpu.make_async_copy(fr_h.at[pl.ds(0, CH)], fr2.at[b],
                                  sem_l).wait()
            pltpu.make_async_copy(p_h.at[pl.ds(0, CH)], p2.at[b],
                                  sem_l).wait()

        hbase = sid * T_HIST
        load2(0, 0, hbase)

        def hist_chunk(k, _):
            b = lax.rem(k, 2)

            @pl.when(k >= 2)
            def _():
                pltpu.make_async_copy(ones, hist.at[idxv2.at[b]],
                                      sem_s).wait()
            wait2(b)

            @pl.when(k + 1 < NCH_H)
            def _():
                load2(k + 1, 1 - b, hbase)

            @plsc.parallel_loop(0, CH // LN, step=1, unroll=8)
            def _(j):
                fj = fr2[b, pl.ds(j * LN, LN)]
                pj = p2[b, pl.ds(j * LN, LN)]
                idxv2[b, pl.ds(j * LN, LN)] = pj * n + fj
            pltpu.async_copy(ones, hist.at[idxv2.at[b]], sem_s, add=True)
            return 0
        lax.fori_loop(0, NCH_H, hist_chunk, 0)
        for b in range(2):
            pltpu.make_async_copy(ones, hist.at[idxv2.at[b]], sem_s).wait()
        plsc.subcore_barrier()

        ebase = wid * T_EDGE
        load2(0, 0, ebase)

        def val_chunk(k, _):
            b = lax.rem(k, 2)

            @pl.when(k >= 2)
            def _():
                pltpu.make_async_copy(vals2.at[b],
                                      vals_h.at[pl.ds(0, CH)], sem_v).wait()
            wait2(b)

            @pl.when(k + 1 < NCH_L)
            def _():
                load2(k + 1, 1 - b, ebase)

            @plsc.parallel_loop(0, CH // LN, step=1, unroll=8)
            def _(j):
                fj = fr2[b, pl.ds(j * LN, LN)]
                pj = p2[b, pl.ds(j * LN, LN)]
                idxv2[b, pl.ds(j * LN, LN)] = pj * n + fj
            pltpu.sync_copy(hist.at[idxv2.at[b]], vals2.at[b])

            @plsc.parallel_loop(0, CH // LN, step=1, unroll=8)
            def _(j):
                v = vals2[b, pl.ds(j * LN, LN)]
                vals2[b, pl.ds(j * LN, LN)] = 1.0 / v
            pltpu.async_copy(vals2.at[b], vals_h.at[pl.ds(ebase + k * CH, CH)],
                             sem_v)
            return 0
        lax.fori_loop(0, NCH_L, val_chunk, 0)
        for b in range(2):
            pltpu.make_async_copy(vals2.at[b], vals_h.at[pl.ds(0, CH)],
                                  sem_v).wait()

    return pl.kernel(body, out_type=out_type, mesh=mesh,
                     scratch_types=scratch_types,
                     compiler_params=pltpu.CompilerParams(
                         use_tc_tiling_on_sc=False,
                         needs_layout_passes=False))


def _make_layer_pass(cfg, cp, ct):
    n = cfg["n"]
    CH = cfg["ch"]
    NSUB = 5
    SUB = CH // NSUB
    T_EDGE = cfg["t_edge"]
    NCH_L = T_EDGE // CH
    HROWS = cfg["hrows"]
    HR_T = HROWS // NS
    ZR = cfg["zr"]
    emb = cfg["emb"]

    mesh = plsc.VectorSubcoreMesh(
        core_axis_name="c", subcore_axis_name="s", num_cores=NC,
        num_subcores=NS)
    out_type = [jax.ShapeDtypeStruct((NC, HROWS, emb), jnp.float32)]
    scratch_types = [
        pltpu.VMEM_SHARED((HROWS, emb), jnp.float32),
        pltpu.VMEM((ZR, emb), jnp.float32),
        pltpu.VMEM((2, CH), jnp.int32),
        pltpu.VMEM((2, CH), jnp.int32),
        pltpu.VMEM((2, CH), jnp.int32),
        pltpu.VMEM((2, CH), jnp.float32),
        pltpu.VMEM((2, NSUB, SUB), jnp.int32),
        pltpu.VMEM((2, NSUB, SUB), jnp.int32),
        pltpu.VMEM((3, SUB, emb), jnp.float32),
        pltpu.SemaphoreType.DMA,
        pltpu.SemaphoreType.DMA,
        pltpu.SemaphoreType.DMA,
    ]

    def body(fr_h, p_h, to_h, vals_in_h, tab_h, part_h,
             acc, zrows, fr2, p2, to2, vals2, idxw2, idxf2, rows2,
             sem_l, sem_g, sem_s):
        cid = lax.axis_index("c")
        sid = lax.axis_index("s")
        wid = cid * NS + sid

        def zr_body(i, _):
            zrows[i] = jnp.zeros((LN,), jnp.float32)
            return 0
        lax.fori_loop(0, ZR, zr_body, 0)
        for k in range(HR_T // ZR):
            pltpu.async_copy(zrows, acc.at[pl.ds(sid * HR_T + k * ZR, ZR)],
                             sem_s)
        for k in range(HR_T // ZR):
            pltpu.make_async_copy(
                zrows, acc.at[pl.ds(sid * HR_T + k * ZR, ZR)], sem_s).wait()
        plsc.subcore_barrier()

        def load4(k, b, base):
            for s, d in ((fr_h, fr2), (p_h, p2), (to_h, to2),
                         (vals_in_h, vals2)):
                pltpu.async_copy(s.at[pl.ds(base + k * CH, CH)],
                                 d.at[b], sem_l)

        def wait4(b):
            for s, d in ((fr_h, fr2), (p_h, p2), (to_h, to2),
                         (vals_in_h, vals2)):
                pltpu.make_async_copy(s.at[pl.ds(0, CH)], d.at[b],
                                      sem_l).wait()

        ebase = wid * T_EDGE
        load4(0, 0, ebase)

        def edge_chunk(k, _):
            b = lax.rem(k, 2)
            wait4(b)

            @pl.when(k + 1 < NCH_L)
            def _():
                load4(k + 1, 1 - b, ebase)

            @plsc.parallel_loop(0, CH // LN, step=1, unroll=8)
            def _(j):
                s = j // (SUB // LN)
                col = (j % (SUB // LN)) * LN
                fj = fr2[b, pl.ds(j * LN, LN)]
                pj = p2[b, pl.ds(j * LN, LN)]
                tj = to2[b, pl.ds(j * LN, LN)]
                idxw2[b, s, pl.ds(col, LN)] = pj * cp + tj * ct
                idxf2[b, s, pl.ds(col, LN)] = fj

            @pl.when(k >= 1)
            def _():
                pltpu.make_async_copy(
                    rows2.at[lax.rem(k * NSUB, 3)],
                    acc.at[idxf2.at[1 - b, 2]], sem_s).wait()
            pltpu.async_copy(tab_h.at[idxw2.at[b, 0]],
                             rows2.at[lax.rem(k * NSUB, 3)], sem_g)
            for s in range(NSUB):
                g_mod3 = lax.rem(k * NSUB + s, 3)
                nxt_mod3 = lax.rem(k * NSUB + s + 1, 3)
                if s < NSUB - 1:
                    if s >= 2:
                        pltpu.make_async_copy(
                            rows2.at[nxt_mod3],
                            acc.at[idxf2.at[b, s - 2]], sem_s).wait()
                    else:
                        @pl.when(k >= 1)
                        def _():
                            pltpu.make_async_copy(
                                rows2.at[nxt_mod3],
                                acc.at[idxf2.at[1 - b, s + 3]],
                                sem_s).wait()
                    pltpu.async_copy(tab_h.at[idxw2.at[b, s + 1]],
                                     rows2.at[nxt_mod3], sem_g)
                pltpu.make_async_copy(tab_h.at[idxw2.at[b, s]],
                                      rows2.at[g_mod3], sem_g).wait()

                @plsc.parallel_loop(0, SUB, step=1, unroll=8)
                def _(j):
                    sp = plsc.load_gather(
                        vals2, [jnp.full((LN,), b, jnp.int32),
                                jnp.full((LN,), s * SUB + j, jnp.int32)])
                    rows2[g_mod3, j] = rows2[g_mod3, j] * sp
                pltpu.async_copy(rows2.at[g_mod3], acc.at[idxf2.at[b, s]],
                                 sem_s, add=True)
            return 0
        lax.fori_loop(0, NCH_L, edge_chunk, 0)
        bl = (NCH_L - 1) % 2
        for s in (2, 3, 4):
            g = (NCH_L - 1) * NSUB + s
            pltpu.make_async_copy(rows2.at[g % 3],
                                  acc.at[idxf2.at[bl, s]], sem_s).wait()

        plsc.subcore_barrier()
        pltpu.sync_copy(acc.at[pl.ds(sid * HR_T, HR_T)],
                        part_h.at[cid].at[pl.ds(sid * HR_T, HR_T)])

    return pl.kernel(body, out_type=out_type, mesh=mesh,
                     scratch_types=scratch_types,
                     compiler_params=pltpu.CompilerParams(
                         use_tc_tiling_on_sc=False,
                         needs_layout_passes=False))


def _tc_hw2(n_real, r, emb, ncls, blk):
    grid = n_real // blk

    def body(hp_ref, b1_ref, w2_ref, out_ref):
        h = jax.nn.relu(hp_ref[0] + hp_ref[1] + b1_ref[0][None, :])
        out_ref[...] = jnp.dot(h, w2_ref[...],
                               preferred_element_type=jnp.float32)

    return pl.pallas_call(
        body,
        grid=(grid,),
        in_specs=[
            pl.BlockSpec((NC, blk, emb), lambda i: (0, i, 0)),
            pl.BlockSpec((1, emb), lambda i: (0, 0)),
            pl.BlockSpec((emb, r * ncls), lambda i: (0, 0)),
        ],
        out_specs=pl.BlockSpec((blk, r * ncls), lambda i: (i, 0)),
        out_shape=jax.ShapeDtypeStruct((n_real, r * ncls), jnp.float32),
    )


def _tc_combine(n_real, ncls, blk):
    grid = n_real // blk

    def body(op_ref, b2_ref, out_ref):
        out_ref[...] = op_ref[0] + op_ref[1] + b2_ref[0][None, :]

    return pl.pallas_call(
        body,
        grid=(grid,),
        in_specs=[
            pl.BlockSpec((NC, blk, ncls), lambda i: (0, i, 0)),
            pl.BlockSpec((1, ncls), lambda i: (0, 0)),
        ],
        out_specs=pl.BlockSpec((blk, ncls), lambda i: (i, 0)),
        out_shape=jax.ShapeDtypeStruct((n_real, ncls), jnp.float32),
    )


def _rgcn(triples, weights1, weights2, bias1, bias2, cfg):
    n, r = cfg["n"], cfg["r"]
    emb, ncls = cfg["emb"], cfg["ncls"]

    w1_flat = weights1.reshape(r * n, emb)
    fr = triples[:, 0]
    p = triples[:, 1]
    to = triples[:, 2]

    (vals,) = _make_vals_pass(cfg)(fr, p)

    (hpart,) = _make_layer_pass(cfg, n, 1)(fr, p, to, vals, w1_flat)

    w2cat = jnp.transpose(weights2, (1, 0, 2)).reshape(emb, r * ncls)
    hw2 = _tc_hw2(n, r, emb, ncls, cfg["tc_blk"])(
        hpart, bias1.reshape(1, emb), w2cat)
    hw2_flat = hw2.reshape(n * r, ncls)

    (opart,) = _make_layer_pass(cfg, 1, r)(fr, p, to, vals, hw2_flat)

    out = _tc_combine(n, ncls, cfg["tc_blk"])(opart, bias2.reshape(1, ncls))
    return out


_CFG_FULL = dict(
    n=50000, r=8, emb=16, ncls=16,
    ch=2000, t_edge=50000,
    hbins=409600, hrows=50000,
    zf=3200, zr=125,
    tc_blk=2000,
)


def kernel(triples, weights1, weights2, bias1, bias2):
    return _rgcn(triples, weights1, weights2, bias1, bias2, _CFG_FULL)
